# Initial kernel scaffold; baseline (speedup 1.0000x reference)
import jax
import jax.numpy as jnp
from jax.experimental import pallas as pl


def kernel(x_in, f_w, f_b, f_g, f_bn, p_w, p_b, p_g, p_bn, r_w, r_b, r_g, r_bn, e1_w, e1_b, e2_w, e2_b, gamma, beta):
    raise NotImplementedError("write your pallas kernel here")


# speedup vs baseline: 14.6916x; 14.6916x over previous
import numpy as np
import jax
import jax.numpy as jnp
from jax.experimental import pallas as pl
from jax.experimental.pallas import tpu as pltpu

_C = 96
_D4 = 24
_K = 9
_PS = 32
_N = 1024
_WSZ = 7
_PPB = _WSZ * _WSZ
_W = _WSZ * _PS


def _grid_rows() -> np.ndarray:
    gi, gj = np.meshgrid(np.arange(_PS), np.arange(_PS), indexing="ij")
    g = np.stack([gi, gj], axis=-1).astype(np.float32).reshape(_N, 2)
    g = (g - g.mean(0)) / (g.std(0, ddof=1) + 1e-5)
    pad = np.zeros((8, _N), np.float32)
    pad[0] = g[:, 0]
    pad[1] = g[:, 1]
    return pad


def _patch_kernel(x_ref, gp_ref, fw_ref, fb_ref, fg_ref, fbn_ref,
                  e1a_ref, e1b_ref, e1bias_ref, e2w_ref, e2b_ref,
                  gam_ref, bet_ref, pw_ref, pb_ref, rw_ref, rb_ref,
                  yp_ref, yr_ref, st_ref):
    f32 = jnp.float32
    x = x_ref[0]

    r_out = jax.lax.dot(rw_ref[...], x, preferred_element_type=f32) + rb_ref[...]
    yr_ref[0] = r_out

    f = jax.lax.dot(fw_ref[...], x, preferred_element_type=f32) + fb_ref[...]
    mu = jnp.mean(f)
    var = jnp.mean((f - mu) ** 2)
    nodes = (f - mu) * jax.lax.rsqrt(var + 1e-5) * fg_ref[...] + fbn_ref[...]

    xaug = jnp.concatenate([nodes, gp_ref[...]], axis=0)
    nrm = jnp.sqrt(jnp.sum(xaug * xaug, axis=0, keepdims=True))
    xn = xaug / jnp.maximum(nrm, 1e-8)
    sim = jax.lax.dot_general(xn, xn, (((0,), (0,)), ((), ())),
                              preferred_element_type=f32)

    rowi = jax.lax.broadcasted_iota(jnp.int32, (_N, _N), 0)
    coli = jax.lax.broadcasted_iota(jnp.int32, (_N, _N), 1)
    neg = f32(-1e30)
    sim = jnp.where(rowi == coli, neg, sim)

    a_src = jax.lax.dot(e1a_ref[...], nodes, preferred_element_type=f32) + e1bias_ref[...]

    acc = jnp.zeros((_D4, _N), f32)
    for _ in range(_K):
        rowmax = jnp.max(sim, axis=1, keepdims=True)
        idx = jnp.min(jnp.where(sim == rowmax, coli, _N), axis=1, keepdims=True)
        sel = coli == idx
        sel_f = sel.astype(f32)
        nodes_d = jax.lax.dot_general(nodes, sel_f, (((1,), (1,)), ((), ())),
                                      preferred_element_type=f32)
        hdn = jax.nn.silu(a_src + jax.lax.dot(e1b_ref[...], nodes_d,
                                              preferred_element_type=f32))
        wt = jax.nn.sigmoid(jnp.sum(hdn * e2w_ref[...], axis=0, keepdims=True)
                            + e2b_ref[...])
        acc = acc + nodes_d * wt
        sim = jnp.where(sel, neg, sim)

    out_nodes = acc * gam_ref[...] + nodes * bet_ref[...]

    po = jax.lax.dot(pw_ref[...], out_nodes, preferred_element_type=f32) + pb_ref[...]
    yp_ref[0] = po

    def tile(v):
        return jnp.full((2, 128), v, f32)

    st_ref[0] = jnp.concatenate(
        [tile(jnp.sum(po)), tile(jnp.sum(po * po)),
         tile(jnp.sum(r_out)), tile(jnp.sum(r_out * r_out))], axis=0)


def _fin_kernel(yp_ref, yr_ref, ap_ref, ar_ref, cc_ref, o_ref):
    y7 = yp_ref[...] * ap_ref[...] + yr_ref[...] * ar_ref[...] + cc_ref[...]
    band = jnp.concatenate(
        [y7[jj].reshape(_C, _PS, _PS) for jj in range(_WSZ)], axis=2)
    o_ref[0] = band


def _bcast_spec(shape):
    nd = len(shape)
    return pl.BlockSpec(shape, lambda p, _n=nd: (0,) * _n)


def kernel(x_in, f_w, f_b, f_g, f_bn, p_w, p_b, p_g, p_bn, r_w, r_b, r_g,
           r_bn, e1_w, e1_b, e2_w, e2_b, gamma, beta):
    f32 = jnp.float32
    B = x_in.shape[0]
    npatch = B * _PPB

    xw = (x_in.reshape(B, _C, _WSZ, _PS, _WSZ, _PS)
          .transpose(0, 2, 4, 1, 3, 5)
          .reshape(npatch, _C, _N))

    gp = jnp.asarray(_grid_rows())
    e1a = e1_w[:, :_D4]
    e1b_m = e1_w[:, _D4:]

    col = lambda v: v.reshape(-1, 1).astype(f32)

    def bandidx(p):
        return (p // _PPB, 0, (p % _PPB) // _WSZ)

    yp, yr, st = pl.pallas_call(
        _patch_kernel,
        grid=(npatch,),
        in_specs=[
            pl.BlockSpec((1, _C, _N), lambda p: (p, 0, 0)),
            _bcast_spec((8, _N)),
            _bcast_spec((_D4, _C)),
            _bcast_spec((_D4, 1)),
            _bcast_spec((_D4, 1)),
            _bcast_spec((_D4, 1)),
            _bcast_spec((_D4, _D4)),
            _bcast_spec((_D4, _D4)),
            _bcast_spec((_D4, 1)),
            _bcast_spec((_D4, 1)),
            _bcast_spec((1, 1)),
            _bcast_spec((1, 1)),
            _bcast_spec((1, 1)),
            _bcast_spec((_C, _D4)),
            _bcast_spec((_C, 1)),
            _bcast_spec((_C, _C)),
            _bcast_spec((_C, 1)),
        ],
        out_specs=[
            pl.BlockSpec((1, _C, _N), lambda p: (p, 0, 0)),
            pl.BlockSpec((1, _C, _N), lambda p: (p, 0, 0)),
            pl.BlockSpec((1, 8, 128), lambda p: (p, 0, 0)),
        ],
        out_shape=[
            jax.ShapeDtypeStruct((npatch, _C, _N), f32),
            jax.ShapeDtypeStruct((npatch, _C, _N), f32),
            jax.ShapeDtypeStruct((npatch, 8, 128), f32),
        ],
        compiler_params=pltpu.CompilerParams(
            dimension_semantics=("arbitrary",)),
    )(xw, gp, f_w, col(f_b), col(f_g), col(f_bn), e1a, e1b_m, col(e1_b),
      e2_w.reshape(_D4, 1), e2_b.reshape(1, 1), gamma.reshape(1, 1),
      beta.reshape(1, 1), p_w, col(p_b), r_w, col(r_b))

    cnt = f32(_PPB * _C * _N)
    s_p = st[:, 0, 0].reshape(B, _PPB).sum(axis=1)
    ss_p = st[:, 2, 0].reshape(B, _PPB).sum(axis=1)
    s_r = st[:, 4, 0].reshape(B, _PPB).sum(axis=1)
    ss_r = st[:, 6, 0].reshape(B, _PPB).sum(axis=1)
    mu_p = s_p / cnt
    mu_r = s_r / cnt
    inv_p = jax.lax.rsqrt(jnp.maximum(ss_p / cnt - mu_p * mu_p, 0.0) + 1e-5)
    inv_r = jax.lax.rsqrt(jnp.maximum(ss_r / cnt - mu_r * mu_r, 0.0) + 1e-5)

    ap = p_g[None, :] * inv_p[:, None]
    ar = r_g[None, :] * inv_r[:, None]
    cc = (p_bn[None, :] + r_bn[None, :]
          - mu_p[:, None] * ap - mu_r[:, None] * ar)

    out = pl.pallas_call(
        _fin_kernel,
        grid=(B, _WSZ),
        in_specs=[
            pl.BlockSpec((_WSZ, _C, _N), lambda b, i: (b * _WSZ + i, 0, 0)),
            pl.BlockSpec((_WSZ, _C, _N), lambda b, i: (b * _WSZ + i, 0, 0)),
            pl.BlockSpec((1, _C, 1), lambda b, i: (b, 0, 0)),
            pl.BlockSpec((1, _C, 1), lambda b, i: (b, 0, 0)),
            pl.BlockSpec((1, _C, 1), lambda b, i: (b, 0, 0)),
        ],
        out_specs=pl.BlockSpec((1, _C, _PS, _W), lambda b, i: (b, 0, i, 0)),
        out_shape=jax.ShapeDtypeStruct((B, _C, _W, _W), f32),
        compiler_params=pltpu.CompilerParams(
            dimension_semantics=("arbitrary", "arbitrary")),
    )(yp, yr, ap[:, :, None], ar[:, :, None], cc[:, :, None])

    return out


# speedup vs baseline: 16.2121x; 1.1035x over previous
import numpy as np
import jax
import jax.numpy as jnp
from jax.experimental import pallas as pl
from jax.experimental.pallas import tpu as pltpu

_C = 96
_D4 = 24
_K = 9
_PS = 32
_N = 1024
_WSZ = 7
_PPB = _WSZ * _WSZ
_W = _WSZ * _PS


def _grid_rows() -> np.ndarray:
    gi, gj = np.meshgrid(np.arange(_PS), np.arange(_PS), indexing="ij")
    g = np.stack([gi, gj], axis=-1).astype(np.float32).reshape(_N, 2)
    g = (g - g.mean(0)) / (g.std(0, ddof=1) + 1e-5)
    pad = np.zeros((8, _N), np.float32)
    pad[0] = g[:, 0]
    pad[1] = g[:, 1]
    return pad


def _patch_kernel(x_ref, gp_ref, fw_ref, fb_ref, fg_ref, fbn_ref,
                  e1a_ref, e1b_ref, e1bias_ref, e2w_ref, e2b_ref,
                  gam_ref, bet_ref, pw_ref, pb_ref, rw_ref, rb_ref,
                  yp_ref, yr_ref, st_ref):
    f32 = jnp.float32
    x = x_ref[0]

    r_out = jax.lax.dot(rw_ref[...], x, preferred_element_type=f32) + rb_ref[...]
    yr_ref[0] = r_out

    f = jax.lax.dot(fw_ref[...], x, preferred_element_type=f32) + fb_ref[...]
    mu = jnp.mean(f)
    var = jnp.mean((f - mu) ** 2)
    nodes = (f - mu) * jax.lax.rsqrt(var + 1e-5) * fg_ref[...] + fbn_ref[...]

    xaug = jnp.concatenate([nodes, gp_ref[...]], axis=0)
    nrm = jnp.sqrt(jnp.sum(xaug * xaug, axis=0, keepdims=True))
    xn = xaug / jnp.maximum(nrm, 1e-8)
    sim = jax.lax.dot_general(xn, xn, (((0,), (0,)), ((), ())),
                              preferred_element_type=f32)

    rowi = jax.lax.broadcasted_iota(jnp.int32, (_N, _N), 0)
    coli = jax.lax.broadcasted_iota(jnp.int32, (_N, _N), 1)
    neg = f32(-1e30)
    sim = jnp.where(rowi == coli, neg, sim)

    a_src = jax.lax.dot(e1a_ref[...], nodes, preferred_element_type=f32) + e1bias_ref[...]

    acc = jnp.zeros((_D4, _N), f32)
    for _ in range(_K):
        idx = jnp.argmax(sim, axis=1, keepdims=True)
        sel = coli == idx
        sel_f = sel.astype(f32)
        nodes_d = jax.lax.dot_general(nodes, sel_f, (((1,), (1,)), ((), ())),
                                      preferred_element_type=f32)
        hdn = jax.nn.silu(a_src + jax.lax.dot(e1b_ref[...], nodes_d,
                                              preferred_element_type=f32))
        wt = jax.nn.sigmoid(jnp.sum(hdn * e2w_ref[...], axis=0, keepdims=True)
                            + e2b_ref[...])
        acc = acc + nodes_d * wt
        sim = jnp.where(sel, neg, sim)

    out_nodes = acc * gam_ref[...] + nodes * bet_ref[...]

    po = jax.lax.dot(pw_ref[...], out_nodes, preferred_element_type=f32) + pb_ref[...]
    yp_ref[0] = po

    def tile(v):
        return jnp.full((2, 128), v, f32)

    st_ref[0] = jnp.concatenate(
        [tile(jnp.sum(po)), tile(jnp.sum(po * po)),
         tile(jnp.sum(r_out)), tile(jnp.sum(r_out * r_out))], axis=0)


def _fin_kernel(yp_ref, yr_ref, ap_ref, ar_ref, cc_ref, o_ref):
    y7 = yp_ref[...] * ap_ref[...] + yr_ref[...] * ar_ref[...] + cc_ref[...]
    band = jnp.concatenate(
        [y7[jj].reshape(_C, _PS, _PS) for jj in range(_WSZ)], axis=2)
    o_ref[0] = band


def _bcast_spec(shape):
    nd = len(shape)
    return pl.BlockSpec(shape, lambda p, _n=nd: (0,) * _n)


def kernel(x_in, f_w, f_b, f_g, f_bn, p_w, p_b, p_g, p_bn, r_w, r_b, r_g,
           r_bn, e1_w, e1_b, e2_w, e2_b, gamma, beta):
    f32 = jnp.float32
    B = x_in.shape[0]
    npatch = B * _PPB

    xw = (x_in.reshape(B, _C, _WSZ, _PS, _WSZ, _PS)
          .transpose(0, 2, 4, 1, 3, 5)
          .reshape(npatch, _C, _N))

    gp = jnp.asarray(_grid_rows())
    e1a = e1_w[:, :_D4]
    e1b_m = e1_w[:, _D4:]

    col = lambda v: v.reshape(-1, 1).astype(f32)

    def bandidx(p):
        return (p // _PPB, 0, (p % _PPB) // _WSZ)

    yp, yr, st = pl.pallas_call(
        _patch_kernel,
        grid=(npatch,),
        in_specs=[
            pl.BlockSpec((1, _C, _N), lambda p: (p, 0, 0)),
            _bcast_spec((8, _N)),
            _bcast_spec((_D4, _C)),
            _bcast_spec((_D4, 1)),
            _bcast_spec((_D4, 1)),
            _bcast_spec((_D4, 1)),
            _bcast_spec((_D4, _D4)),
            _bcast_spec((_D4, _D4)),
            _bcast_spec((_D4, 1)),
            _bcast_spec((_D4, 1)),
            _bcast_spec((1, 1)),
            _bcast_spec((1, 1)),
            _bcast_spec((1, 1)),
            _bcast_spec((_C, _D4)),
            _bcast_spec((_C, 1)),
            _bcast_spec((_C, _C)),
            _bcast_spec((_C, 1)),
        ],
        out_specs=[
            pl.BlockSpec((1, _C, _N), lambda p: (p, 0, 0)),
            pl.BlockSpec((1, _C, _N), lambda p: (p, 0, 0)),
            pl.BlockSpec((1, 8, 128), lambda p: (p, 0, 0)),
        ],
        out_shape=[
            jax.ShapeDtypeStruct((npatch, _C, _N), f32),
            jax.ShapeDtypeStruct((npatch, _C, _N), f32),
            jax.ShapeDtypeStruct((npatch, 8, 128), f32),
        ],
        compiler_params=pltpu.CompilerParams(
            dimension_semantics=("arbitrary",)),
    )(xw, gp, f_w, col(f_b), col(f_g), col(f_bn), e1a, e1b_m, col(e1_b),
      e2_w.reshape(_D4, 1), e2_b.reshape(1, 1), gamma.reshape(1, 1),
      beta.reshape(1, 1), p_w, col(p_b), r_w, col(r_b))

    cnt = f32(_PPB * _C * _N)
    s_p = st[:, 0, 0].reshape(B, _PPB).sum(axis=1)
    ss_p = st[:, 2, 0].reshape(B, _PPB).sum(axis=1)
    s_r = st[:, 4, 0].reshape(B, _PPB).sum(axis=1)
    ss_r = st[:, 6, 0].reshape(B, _PPB).sum(axis=1)
    mu_p = s_p / cnt
    mu_r = s_r / cnt
    inv_p = jax.lax.rsqrt(jnp.maximum(ss_p / cnt - mu_p * mu_p, 0.0) + 1e-5)
    inv_r = jax.lax.rsqrt(jnp.maximum(ss_r / cnt - mu_r * mu_r, 0.0) + 1e-5)

    ap = p_g[None, :] * inv_p[:, None]
    ar = r_g[None, :] * inv_r[:, None]
    cc = (p_bn[None, :] + r_bn[None, :]
          - mu_p[:, None] * ap - mu_r[:, None] * ar)

    out = pl.pallas_call(
        _fin_kernel,
        grid=(B, _WSZ),
        in_specs=[
            pl.BlockSpec((_WSZ, _C, _N), lambda b, i: (b * _WSZ + i, 0, 0)),
            pl.BlockSpec((_WSZ, _C, _N), lambda b, i: (b * _WSZ + i, 0, 0)),
            pl.BlockSpec((1, _C, 1), lambda b, i: (b, 0, 0)),
            pl.BlockSpec((1, _C, 1), lambda b, i: (b, 0, 0)),
            pl.BlockSpec((1, _C, 1), lambda b, i: (b, 0, 0)),
        ],
        out_specs=pl.BlockSpec((1, _C, _PS, _W), lambda b, i: (b, 0, i, 0)),
        out_shape=jax.ShapeDtypeStruct((B, _C, _W, _W), f32),
        compiler_params=pltpu.CompilerParams(
            dimension_semantics=("arbitrary", "arbitrary")),
    )(yp, yr, ap[:, :, None], ar[:, :, None], cc[:, :, None])

    return out


# speedup vs baseline: 18.9508x; 1.1689x over previous
import numpy as np
import jax
import jax.numpy as jnp
from jax.experimental import pallas as pl
from jax.experimental.pallas import tpu as pltpu

_C = 96
_D4 = 24
_K = 9
_PS = 32
_N = 1024
_WSZ = 7
_PPB = _WSZ * _WSZ
_W = _WSZ * _PS


def _grid_rows() -> np.ndarray:
    gi, gj = np.meshgrid(np.arange(_PS), np.arange(_PS), indexing="ij")
    g = np.stack([gi, gj], axis=-1).astype(np.float32).reshape(_N, 2)
    g = (g - g.mean(0)) / (g.std(0, ddof=1) + 1e-5)
    pad = np.zeros((8, _N), np.float32)
    pad[0] = g[:, 0]
    pad[1] = g[:, 1]
    return pad


def _win_kernel(x_ref, xw_ref):
    band = x_ref[0]
    for jj in range(_WSZ):
        xp = jax.lax.slice(band, (0, 0, jj * _PS), (_C, _PS, (jj + 1) * _PS))
        xw_ref[jj] = xp.reshape(_C, _N)


def _patch_kernel(x_ref, gp_ref, fw_ref, fb_ref, fg_ref, fbn_ref,
                  e1a_ref, e1b_ref, e1bias_ref, e2w_ref, e2b_ref,
                  gam_ref, bet_ref, pw_ref, pb_ref, rw_ref, rb_ref,
                  yp_ref, yr_ref, st_ref):
    f32 = jnp.float32
    x = x_ref[0]

    r_out = jax.lax.dot(rw_ref[...], x, preferred_element_type=f32) + rb_ref[...]
    yr_ref[0] = r_out

    f = jax.lax.dot(fw_ref[...], x, preferred_element_type=f32) + fb_ref[...]
    mu = jnp.mean(f)
    var = jnp.mean((f - mu) ** 2)
    nodes = (f - mu) * jax.lax.rsqrt(var + 1e-5) * fg_ref[...] + fbn_ref[...]

    xaug = jnp.concatenate([nodes, gp_ref[...]], axis=0)
    nrm = jnp.sqrt(jnp.sum(xaug * xaug, axis=0, keepdims=True))
    xn = xaug / jnp.maximum(nrm, 1e-8)
    sim = jax.lax.dot_general(xn, xn, (((0,), (0,)), ((), ())),
                              preferred_element_type=f32)

    rowi = jax.lax.broadcasted_iota(jnp.int32, (_N, _N), 0)
    coli = jax.lax.broadcasted_iota(jnp.int32, (_N, _N), 1)
    neg = f32(-1e30)
    sim = jnp.where(rowi == coli, neg, sim)

    a_src = jax.lax.dot(e1a_ref[...], nodes, preferred_element_type=f32) + e1bias_ref[...]

    acc = jnp.zeros((_D4, _N), f32)
    for _ in range(_K):
        idx = jnp.argmax(sim, axis=1, keepdims=True)
        sel = coli == idx
        sel_f = sel.astype(f32)
        nodes_d = jax.lax.dot_general(nodes, sel_f, (((1,), (1,)), ((), ())),
                                      preferred_element_type=f32)
        hdn = jax.nn.silu(a_src + jax.lax.dot(e1b_ref[...], nodes_d,
                                              preferred_element_type=f32))
        wt = jax.nn.sigmoid(jnp.sum(hdn * e2w_ref[...], axis=0, keepdims=True)
                            + e2b_ref[...])
        acc = acc + nodes_d * wt
        sim = jnp.where(sel, neg, sim)

    out_nodes = acc * gam_ref[...] + nodes * bet_ref[...]

    po = jax.lax.dot(pw_ref[...], out_nodes, preferred_element_type=f32) + pb_ref[...]
    yp_ref[0] = po

    def tile(v):
        return jnp.full((2, 128), v, f32)

    st_ref[0] = jnp.concatenate(
        [tile(jnp.sum(po)), tile(jnp.sum(po * po)),
         tile(jnp.sum(r_out)), tile(jnp.sum(r_out * r_out))], axis=0)


def _fin_kernel(yp_ref, yr_ref, ap_ref, ar_ref, cc_ref, o_ref):
    y7 = yp_ref[...] * ap_ref[...] + yr_ref[...] * ar_ref[...] + cc_ref[...]
    band = jnp.concatenate(
        [y7[jj].reshape(_C, _PS, _PS) for jj in range(_WSZ)], axis=2)
    o_ref[0] = band


def _bcast_spec(shape):
    nd = len(shape)
    return pl.BlockSpec(shape, lambda p, _n=nd: (0,) * _n)


def kernel(x_in, f_w, f_b, f_g, f_bn, p_w, p_b, p_g, p_bn, r_w, r_b, r_g,
           r_bn, e1_w, e1_b, e2_w, e2_b, gamma, beta):
    f32 = jnp.float32
    B = x_in.shape[0]
    npatch = B * _PPB

    xw = pl.pallas_call(
        _win_kernel,
        grid=(B, _WSZ),
        in_specs=[pl.BlockSpec((1, _C, _PS, _W), lambda b, i: (b, 0, i, 0))],
        out_specs=pl.BlockSpec((_WSZ, _C, _N), lambda b, i: (b * _WSZ + i, 0, 0)),
        out_shape=jax.ShapeDtypeStruct((npatch, _C, _N), f32),
        compiler_params=pltpu.CompilerParams(
            dimension_semantics=("arbitrary", "arbitrary")),
    )(x_in)

    gp = jnp.asarray(_grid_rows())
    e1a = e1_w[:, :_D4]
    e1b_m = e1_w[:, _D4:]

    col = lambda v: v.reshape(-1, 1).astype(f32)

    def bandidx(p):
        return (p // _PPB, 0, (p % _PPB) // _WSZ)

    yp, yr, st = pl.pallas_call(
        _patch_kernel,
        grid=(npatch,),
        in_specs=[
            pl.BlockSpec((1, _C, _N), lambda p: (p, 0, 0)),
            _bcast_spec((8, _N)),
            _bcast_spec((_D4, _C)),
            _bcast_spec((_D4, 1)),
            _bcast_spec((_D4, 1)),
            _bcast_spec((_D4, 1)),
            _bcast_spec((_D4, _D4)),
            _bcast_spec((_D4, _D4)),
            _bcast_spec((_D4, 1)),
            _bcast_spec((_D4, 1)),
            _bcast_spec((1, 1)),
            _bcast_spec((1, 1)),
            _bcast_spec((1, 1)),
            _bcast_spec((_C, _D4)),
            _bcast_spec((_C, 1)),
            _bcast_spec((_C, _C)),
            _bcast_spec((_C, 1)),
        ],
        out_specs=[
            pl.BlockSpec((1, _C, _N), lambda p: (p, 0, 0)),
            pl.BlockSpec((1, _C, _N), lambda p: (p, 0, 0)),
            pl.BlockSpec((1, 8, 128), lambda p: (p, 0, 0)),
        ],
        out_shape=[
            jax.ShapeDtypeStruct((npatch, _C, _N), f32),
            jax.ShapeDtypeStruct((npatch, _C, _N), f32),
            jax.ShapeDtypeStruct((npatch, 8, 128), f32),
        ],
        compiler_params=pltpu.CompilerParams(
            dimension_semantics=("arbitrary",)),
    )(xw, gp, f_w, col(f_b), col(f_g), col(f_bn), e1a, e1b_m, col(e1_b),
      e2_w.reshape(_D4, 1), e2_b.reshape(1, 1), gamma.reshape(1, 1),
      beta.reshape(1, 1), p_w, col(p_b), r_w, col(r_b))

    cnt = f32(_PPB * _C * _N)
    s_p = st[:, 0, 0].reshape(B, _PPB).sum(axis=1)
    ss_p = st[:, 2, 0].reshape(B, _PPB).sum(axis=1)
    s_r = st[:, 4, 0].reshape(B, _PPB).sum(axis=1)
    ss_r = st[:, 6, 0].reshape(B, _PPB).sum(axis=1)
    mu_p = s_p / cnt
    mu_r = s_r / cnt
    inv_p = jax.lax.rsqrt(jnp.maximum(ss_p / cnt - mu_p * mu_p, 0.0) + 1e-5)
    inv_r = jax.lax.rsqrt(jnp.maximum(ss_r / cnt - mu_r * mu_r, 0.0) + 1e-5)

    ap = p_g[None, :] * inv_p[:, None]
    ar = r_g[None, :] * inv_r[:, None]
    cc = (p_bn[None, :] + r_bn[None, :]
          - mu_p[:, None] * ap - mu_r[:, None] * ar)

    out = pl.pallas_call(
        _fin_kernel,
        grid=(B, _WSZ),
        in_specs=[
            pl.BlockSpec((_WSZ, _C, _N), lambda b, i: (b * _WSZ + i, 0, 0)),
            pl.BlockSpec((_WSZ, _C, _N), lambda b, i: (b * _WSZ + i, 0, 0)),
            pl.BlockSpec((1, _C, 1), lambda b, i: (b, 0, 0)),
            pl.BlockSpec((1, _C, 1), lambda b, i: (b, 0, 0)),
            pl.BlockSpec((1, _C, 1), lambda b, i: (b, 0, 0)),
        ],
        out_specs=pl.BlockSpec((1, _C, _PS, _W), lambda b, i: (b, 0, i, 0)),
        out_shape=jax.ShapeDtypeStruct((B, _C, _W, _W), f32),
        compiler_params=pltpu.CompilerParams(
            dimension_semantics=("arbitrary", "arbitrary")),
    )(yp, yr, ap[:, :, None], ar[:, :, None], cc[:, :, None])

    return out
